# Initial kernel scaffold; baseline (speedup 1.0000x reference)
import jax
import jax.numpy as jnp
from jax.experimental import pallas as pl


def kernel(x, edge_index, pos_edge_index, neg_edge_index, W_self1, W_neigh1, b1, W_self2, W_neigh2, b2, W_self3, W_neigh3, b3, P1_W, P1_b, P2_W, P2_b, P3_W, P3_b):
    raise NotImplementedError("write your pallas kernel here")


# speedup vs baseline: 4.8066x; 4.8066x over previous
import jax
import jax.numpy as jnp
from jax import lax
from jax.experimental import pallas as pl
from jax.experimental.pallas import tpu as pltpu
from jax.experimental.pallas import tpu_sc as plsc

N_NODES = 10000
D = 128
E = 320000
NC = 2
NS = 16
NW = NC * NS
CH = 128
NCHUNK = E // CH
N_PAD = 10240
ROWS_PT = N_PAD // NS
DEGW = 128
N_PAIRS = 16384
P_TOT = 2 * N_PAIRS
PCH = 128
PNCH = P_TOT // (NW * PCH)


def _agg_body(h_hbm, src_hbm, dst_hbm, agg_out, sidx, didx, rows, agg_sh, sem):
  c = lax.axis_index("c")
  s = lax.axis_index("s")
  t = s * NC + c
  zero16 = jnp.zeros((16,), jnp.float32)

  def zb(i, carry):
    rows[i // 8, pl.ds((i % 8) * 16, 16)] = zero16
    return carry
  lax.fori_loop(0, CH * 8, zb, 0)

  def za(i, carry):
    pltpu.sync_copy(rows, agg_sh.at[pl.ds(s * ROWS_PT + i * CH, CH)])
    return carry
  lax.fori_loop(0, ROWS_PT // CH, za, 0)

  plsc.subcore_barrier()

  nch = NCHUNK // NW + jnp.where(t < NCHUNK % NW, 1, 0)

  def chunk(i, carry):
    base = (t + i * NW) * CH
    pltpu.sync_copy(src_hbm.at[pl.ds(base, CH)], sidx)
    pltpu.sync_copy(dst_hbm.at[pl.ds(base, CH)], didx)
    pltpu.async_copy(h_hbm.at[sidx], rows, sem).wait()
    pltpu.sync_copy(rows, agg_sh.at[didx], add=True)
    return carry
  lax.fori_loop(0, nch, chunk, 0)

  plsc.subcore_barrier()

  def co(i, carry):
    r0 = s * ROWS_PT + i * CH
    pltpu.sync_copy(agg_sh.at[pl.ds(r0, CH)], rows)
    pltpu.sync_copy(rows, agg_out.at[pl.ds(c * N_PAD + r0, CH)])
    return carry
  lax.fori_loop(0, ROWS_PT // CH, co, 0)


_agg = pl.kernel(
    _agg_body,
    out_type=[jax.ShapeDtypeStruct((NC * N_PAD, D), jnp.float32)],
    mesh=plsc.VectorSubcoreMesh(core_axis_name="c", subcore_axis_name="s"),
    scratch_types=[
        pltpu.VMEM((CH,), jnp.int32),
        pltpu.VMEM((CH,), jnp.int32),
        pltpu.VMEM((CH, D), jnp.float32),
        pltpu.VMEM_SHARED((N_PAD, D), jnp.float32),
        pltpu.SemaphoreType.DMA,
    ],
)


def _deg_body(dst_hbm, deg_out, didx, ones, deg_sh):
  c = lax.axis_index("c")
  s = lax.axis_index("s")
  t = s * NC + c
  zero16 = jnp.zeros((16,), jnp.float32)

  def db(i, carry):
    ones[i // 8, pl.ds((i % 8) * 16, 16)] = zero16
    return carry
  lax.fori_loop(0, CH * 8, db, 0)

  def zd(i, carry):
    pltpu.sync_copy(ones, deg_sh.at[pl.ds(s * ROWS_PT + i * CH, CH)])
    return carry
  lax.fori_loop(0, ROWS_PT // CH, zd, 0)

  one16 = jnp.full((16,), 1.0, jnp.float32)

  def ob(i, carry):
    ones[i // 8, pl.ds((i % 8) * 16, 16)] = one16
    return carry
  lax.fori_loop(0, CH * 8, ob, 0)

  plsc.subcore_barrier()

  nch = NCHUNK // NW + jnp.where(t < NCHUNK % NW, 1, 0)

  def chunk(i, carry):
    base = (t + i * NW) * CH
    pltpu.sync_copy(dst_hbm.at[pl.ds(base, CH)], didx)
    pltpu.sync_copy(ones, deg_sh.at[didx], add=True)
    return carry
  lax.fori_loop(0, nch, chunk, 0)

  plsc.subcore_barrier()

  def cd(i, carry):
    r0 = s * ROWS_PT + i * CH
    pltpu.sync_copy(deg_sh.at[pl.ds(r0, CH)], ones)
    pltpu.sync_copy(ones, deg_out.at[pl.ds(c * N_PAD + r0, CH)])
    return carry
  lax.fori_loop(0, ROWS_PT // CH, cd, 0)


_deg = pl.kernel(
    _deg_body,
    out_type=[jax.ShapeDtypeStruct((NC * N_PAD, DEGW), jnp.float32)],
    mesh=plsc.VectorSubcoreMesh(core_axis_name="c", subcore_axis_name="s"),
    scratch_types=[
        pltpu.VMEM((CH,), jnp.int32),
        pltpu.VMEM((CH, DEGW), jnp.float32),
        pltpu.VMEM_SHARED((N_PAD, DEGW), jnp.float32),
    ],
)


_pairs_mesh = plsc.VectorSubcoreMesh(core_axis_name="c", subcore_axis_name="s")


def _pairs_body(h_hbm, a_hbm, b_hbm, out_hbm, aidx, bidx, ra, rb, sem, sem2):
  c = lax.axis_index("c")
  s = lax.axis_index("s")
  t = s * NC + c

  def chunk(i, carry):
    base = (t * PNCH + i) * PCH
    pltpu.sync_copy(a_hbm.at[pl.ds(base, PCH)], aidx)
    pltpu.sync_copy(b_hbm.at[pl.ds(base, PCH)], bidx)
    cp1 = pltpu.async_copy(h_hbm.at[aidx], ra, sem)
    cp2 = pltpu.async_copy(h_hbm.at[bidx], rb, sem2)
    cp1.wait()
    cp2.wait()

    def mul(j, carry2):
      r = j // 8
      o = (j % 8) * 16
      ra[r, pl.ds(o, 16)] = ra[r, pl.ds(o, 16)] * rb[r, pl.ds(o, 16)]
      return carry2
    lax.fori_loop(0, PCH * 8, mul, 0)
    pltpu.sync_copy(ra, out_hbm.at[pl.ds(base, PCH)])
    return carry
  lax.fori_loop(0, PNCH, chunk, 0)


_pairs = pl.kernel(
    _pairs_body,
    out_type=[jax.ShapeDtypeStruct((P_TOT, D), jnp.float32)],
    mesh=_pairs_mesh,
    scratch_types=[
        pltpu.VMEM((PCH,), jnp.int32),
        pltpu.VMEM((PCH,), jnp.int32),
        pltpu.VMEM((PCH, D), jnp.float32),
        pltpu.VMEM((PCH, D), jnp.float32),
        pltpu.SemaphoreType.DMA,
        pltpu.SemaphoreType.DMA,
    ],
)


def _layer_tc(h, parts, deg2, Ws, Wn, b, relu):
  n = h.shape[0]
  bm = 1000

  def body(h_ref, p_ref, d_ref, ws_ref, wn_ref, b_ref, o_ref):
    dcol = d_ref[0, :, 0:1] + d_ref[1, :, 0:1]
    hn = (p_ref[0] + p_ref[1]) / jnp.maximum(dcol, 1.0)
    acc = jnp.dot(h_ref[...], ws_ref[...], preferred_element_type=jnp.float32)
    acc = acc + jnp.dot(hn, wn_ref[...], preferred_element_type=jnp.float32)
    acc = acc + b_ref[...]
    if relu:
      acc = jnp.maximum(acc, 0.0)
    o_ref[...] = acc

  return pl.pallas_call(
      body,
      grid=(n // bm,),
      in_specs=[
          pl.BlockSpec((bm, D), lambda i: (i, 0)),
          pl.BlockSpec((NC, bm, D), lambda i: (0, i, 0)),
          pl.BlockSpec((NC, bm, DEGW), lambda i: (0, i, 0)),
          pl.BlockSpec((D, D), lambda i: (0, 0)),
          pl.BlockSpec((D, D), lambda i: (0, 0)),
          pl.BlockSpec((1, D), lambda i: (0, 0)),
      ],
      out_specs=pl.BlockSpec((bm, D), lambda i: (i, 0)),
      out_shape=jax.ShapeDtypeStruct((n, D), jnp.float32),
  )(h, parts, deg2, Ws, Wn, b)


def _pred_tc(prod, W1, c1, W2, c2, W3p, c3p):
  m = prod.shape[0]
  bm = 4096

  def body(x_ref, w1, b1, w2, b2, w3, b3, o_ref):
    h1 = jnp.dot(x_ref[...], w1[...], preferred_element_type=jnp.float32)
    h1 = jnp.maximum(h1 + b1[...], 0.0)
    h2 = jnp.dot(h1, w2[...], preferred_element_type=jnp.float32)
    h2 = jnp.maximum(h2 + b2[...], 0.0)
    o_ref[...] = jnp.dot(h2, w3[...], preferred_element_type=jnp.float32) + b3[...]

  return pl.pallas_call(
      body,
      grid=(m // bm,),
      in_specs=[
          pl.BlockSpec((bm, D), lambda i: (i, 0)),
          pl.BlockSpec((D, D), lambda i: (0, 0)),
          pl.BlockSpec((1, D), lambda i: (0, 0)),
          pl.BlockSpec((D, D), lambda i: (0, 0)),
          pl.BlockSpec((1, D), lambda i: (0, 0)),
          pl.BlockSpec((D, 8), lambda i: (0, 0)),
          pl.BlockSpec((1, 8), lambda i: (0, 0)),
      ],
      out_specs=pl.BlockSpec((bm, 8), lambda i: (i, 0)),
      out_shape=jax.ShapeDtypeStruct((m, 8), jnp.float32),
  )(prod, W1, c1, W2, c2, W3p, c3p)


def kernel(x, edge_index, pos_edge_index, neg_edge_index,
           W_self1, W_neigh1, b1, W_self2, W_neigh2, b2,
           W_self3, W_neigh3, b3,
           P1_W, P1_b, P2_W, P2_b, P3_W, P3_b):
  src = edge_index[0].astype(jnp.int32)
  dst = edge_index[1].astype(jnp.int32)

  parts1, = _agg(x, src, dst)
  parts1 = parts1.reshape(NC, N_PAD, D)
  deg2, = _deg(dst)
  deg2 = deg2.reshape(NC, N_PAD, DEGW)
  h1 = _layer_tc(x, parts1, deg2, W_self1, W_neigh1,
                 b1.reshape(1, D), relu=True)
  parts2, = _agg(h1, src, dst)
  parts2 = parts2.reshape(NC, N_PAD, D)
  h2 = _layer_tc(h1, parts2, deg2, W_self2, W_neigh2,
                 b2.reshape(1, D), relu=True)
  parts3, = _agg(h2, src, dst)
  parts3 = parts3.reshape(NC, N_PAD, D)
  h3 = _layer_tc(h2, parts3, deg2, W_self3, W_neigh3,
                 b3.reshape(1, D), relu=False)

  pair = jnp.concatenate([pos_edge_index, neg_edge_index], axis=1)
  a_idx = pair[0].astype(jnp.int32)
  b_idx = pair[1].astype(jnp.int32)
  prod, = _pairs(h3, a_idx, b_idx)

  W3p = jnp.pad(P3_W, ((0, 0), (0, 7)))
  c3p = jnp.pad(P3_b, (0, 7)).reshape(1, 8)
  out = _pred_tc(prod, P1_W, P1_b.reshape(1, D), P2_W, P2_b.reshape(1, D),
                 W3p, c3p)
  col = out[:, 0:1]
  return (col[:N_PAIRS], col[N_PAIRS:])


# speedup vs baseline: 6.6130x; 1.3758x over previous
import jax
import jax.numpy as jnp
from jax import lax
from jax.experimental import pallas as pl
from jax.experimental.pallas import tpu as pltpu
from jax.experimental.pallas import tpu_sc as plsc

N_NODES = 10000
D = 128
E = 320000
NC = 2
NS = 16
NW = NC * NS
CH = 128
NCHUNK = E // CH
N_PAD = 10240
ROWS_PT = N_PAD // NS
DEGW = 128
N_PAIRS = 16384
P_TOT = 2 * N_PAIRS
PCH = 128
PNCH = P_TOT // (NW * PCH)


G_FULL = (NCHUNK // NW) // 2 * 2
G_ITERS = G_FULL // 2


def _agg_body(h_hbm, src_hbm, dst_hbm, agg_out,
              sidx0, sidx1, didx0, didx1, rows0, rows1, agg_sh,
              gsem0, gsem1, ssem0, ssem1):
  c = lax.axis_index("c")
  s = lax.axis_index("s")
  t = s * NC + c
  zero16 = jnp.zeros((16,), jnp.float32)
  sidx = (sidx0, sidx1)
  didx = (didx0, didx1)
  rows = (rows0, rows1)
  gsem = (gsem0, gsem1)
  ssem = (ssem0, ssem1)

  def zb(i, carry):
    rows0[i // 8, pl.ds((i % 8) * 16, 16)] = zero16
    return carry
  lax.fori_loop(0, CH * 8, zb, 0)

  def za(i, carry):
    pltpu.sync_copy(rows0, agg_sh.at[pl.ds(s * ROWS_PT + i * CH, CH)])
    return carry
  lax.fori_loop(0, ROWS_PT // CH, za, 0)

  plsc.subcore_barrier()

  for b in range(2):
    base = (t + b * NW) * CH
    pltpu.sync_copy(src_hbm.at[pl.ds(base, CH)], sidx[b])
    pltpu.sync_copy(dst_hbm.at[pl.ds(base, CH)], didx[b])
    pltpu.async_copy(h_hbm.at[sidx[b]], rows[b], gsem[b])

  def ring(g, carry):
    for b in range(2):
      pltpu.make_async_copy(h_hbm.at[pl.ds(0, CH)], rows[b], gsem[b]).wait()
      pltpu.async_copy(rows[b], agg_sh.at[didx[b]], ssem[b], add=True)
    for b in range(2):
      pltpu.make_async_copy(rows[b], agg_sh.at[didx[b]], ssem[b]).wait()
      nxt = (g + 1) * 2 + b

      @pl.when(nxt < G_FULL)
      def _prefetch():
        base = (t + nxt * NW) * CH
        pltpu.sync_copy(src_hbm.at[pl.ds(base, CH)], sidx[b])
        pltpu.sync_copy(dst_hbm.at[pl.ds(base, CH)], didx[b])
        pltpu.async_copy(h_hbm.at[sidx[b]], rows[b], gsem[b])
    return carry
  lax.fori_loop(0, G_ITERS, ring, 0)

  @pl.when(t < NCHUNK - G_FULL * NW)
  def _tail():
    base = (t + G_FULL * NW) * CH
    pltpu.sync_copy(src_hbm.at[pl.ds(base, CH)], sidx0)
    pltpu.sync_copy(dst_hbm.at[pl.ds(base, CH)], didx0)
    pltpu.async_copy(h_hbm.at[sidx0], rows0, gsem0).wait()
    pltpu.sync_copy(rows0, agg_sh.at[didx0], add=True)

  plsc.subcore_barrier()

  def co(i, carry):
    r0 = s * ROWS_PT + i * CH
    pltpu.sync_copy(agg_sh.at[pl.ds(r0, CH)], rows0)
    pltpu.sync_copy(rows0, agg_out.at[pl.ds(c * N_PAD + r0, CH)])
    return carry
  lax.fori_loop(0, ROWS_PT // CH, co, 0)


_agg = pl.kernel(
    _agg_body,
    out_type=[jax.ShapeDtypeStruct((NC * N_PAD, D), jnp.float32)],
    mesh=plsc.VectorSubcoreMesh(core_axis_name="c", subcore_axis_name="s"),
    scratch_types=[
        pltpu.VMEM((CH,), jnp.int32),
        pltpu.VMEM((CH,), jnp.int32),
        pltpu.VMEM((CH,), jnp.int32),
        pltpu.VMEM((CH,), jnp.int32),
        pltpu.VMEM((CH, D), jnp.float32),
        pltpu.VMEM((CH, D), jnp.float32),
        pltpu.VMEM_SHARED((N_PAD, D), jnp.float32),
        pltpu.SemaphoreType.DMA,
        pltpu.SemaphoreType.DMA,
        pltpu.SemaphoreType.DMA,
        pltpu.SemaphoreType.DMA,
    ],
)


def _deg_body(dst_hbm, deg_out, didx, ones, deg_sh):
  c = lax.axis_index("c")
  s = lax.axis_index("s")
  t = s * NC + c
  zero16 = jnp.zeros((16,), jnp.float32)

  def db(i, carry):
    ones[i // 8, pl.ds((i % 8) * 16, 16)] = zero16
    return carry
  lax.fori_loop(0, CH * 8, db, 0)

  def zd(i, carry):
    pltpu.sync_copy(ones, deg_sh.at[pl.ds(s * ROWS_PT + i * CH, CH)])
    return carry
  lax.fori_loop(0, ROWS_PT // CH, zd, 0)

  one16 = jnp.full((16,), 1.0, jnp.float32)

  def ob(i, carry):
    ones[i // 8, pl.ds((i % 8) * 16, 16)] = one16
    return carry
  lax.fori_loop(0, CH * 8, ob, 0)

  plsc.subcore_barrier()

  nch = NCHUNK // NW + jnp.where(t < NCHUNK % NW, 1, 0)

  def chunk(i, carry):
    base = (t + i * NW) * CH
    pltpu.sync_copy(dst_hbm.at[pl.ds(base, CH)], didx)
    pltpu.sync_copy(ones, deg_sh.at[didx], add=True)
    return carry
  lax.fori_loop(0, nch, chunk, 0)

  plsc.subcore_barrier()

  def cd(i, carry):
    r0 = s * ROWS_PT + i * CH
    pltpu.sync_copy(deg_sh.at[pl.ds(r0, CH)], ones)
    pltpu.sync_copy(ones, deg_out.at[pl.ds(c * N_PAD + r0, CH)])
    return carry
  lax.fori_loop(0, ROWS_PT // CH, cd, 0)


_deg = pl.kernel(
    _deg_body,
    out_type=[jax.ShapeDtypeStruct((NC * N_PAD, DEGW), jnp.float32)],
    mesh=plsc.VectorSubcoreMesh(core_axis_name="c", subcore_axis_name="s"),
    scratch_types=[
        pltpu.VMEM((CH,), jnp.int32),
        pltpu.VMEM((CH, DEGW), jnp.float32),
        pltpu.VMEM_SHARED((N_PAD, DEGW), jnp.float32),
    ],
)


_pairs_mesh = plsc.VectorSubcoreMesh(core_axis_name="c", subcore_axis_name="s")


def _pairs_body(h_hbm, a_hbm, b_hbm, out_hbm, aidx, bidx, ra, rb, sem, sem2):
  c = lax.axis_index("c")
  s = lax.axis_index("s")
  t = s * NC + c

  def chunk(i, carry):
    base = (t * PNCH + i) * PCH
    pltpu.sync_copy(a_hbm.at[pl.ds(base, PCH)], aidx)
    pltpu.sync_copy(b_hbm.at[pl.ds(base, PCH)], bidx)
    cp1 = pltpu.async_copy(h_hbm.at[aidx], ra, sem)
    cp2 = pltpu.async_copy(h_hbm.at[bidx], rb, sem2)
    cp1.wait()
    cp2.wait()

    def mul(j, carry2):
      r = j // 8
      o = (j % 8) * 16
      ra[r, pl.ds(o, 16)] = ra[r, pl.ds(o, 16)] * rb[r, pl.ds(o, 16)]
      return carry2
    lax.fori_loop(0, PCH * 8, mul, 0)
    pltpu.sync_copy(ra, out_hbm.at[pl.ds(base, PCH)])
    return carry
  lax.fori_loop(0, PNCH, chunk, 0)


_pairs = pl.kernel(
    _pairs_body,
    out_type=[jax.ShapeDtypeStruct((P_TOT, D), jnp.float32)],
    mesh=_pairs_mesh,
    scratch_types=[
        pltpu.VMEM((PCH,), jnp.int32),
        pltpu.VMEM((PCH,), jnp.int32),
        pltpu.VMEM((PCH, D), jnp.float32),
        pltpu.VMEM((PCH, D), jnp.float32),
        pltpu.SemaphoreType.DMA,
        pltpu.SemaphoreType.DMA,
    ],
)


def _layer_tc(h, parts, deg2, Ws, Wn, b, relu):
  n = h.shape[0]
  bm = 1000

  def body(h_ref, p_ref, d_ref, ws_ref, wn_ref, b_ref, o_ref):
    dcol = d_ref[0, :, 0:1] + d_ref[1, :, 0:1]
    hn = (p_ref[0] + p_ref[1]) / jnp.maximum(dcol, 1.0)
    acc = jnp.dot(h_ref[...], ws_ref[...], preferred_element_type=jnp.float32)
    acc = acc + jnp.dot(hn, wn_ref[...], preferred_element_type=jnp.float32)
    acc = acc + b_ref[...]
    if relu:
      acc = jnp.maximum(acc, 0.0)
    o_ref[...] = acc

  return pl.pallas_call(
      body,
      grid=(n // bm,),
      in_specs=[
          pl.BlockSpec((bm, D), lambda i: (i, 0)),
          pl.BlockSpec((NC, bm, D), lambda i: (0, i, 0)),
          pl.BlockSpec((NC, bm, DEGW), lambda i: (0, i, 0)),
          pl.BlockSpec((D, D), lambda i: (0, 0)),
          pl.BlockSpec((D, D), lambda i: (0, 0)),
          pl.BlockSpec((1, D), lambda i: (0, 0)),
      ],
      out_specs=pl.BlockSpec((bm, D), lambda i: (i, 0)),
      out_shape=jax.ShapeDtypeStruct((n, D), jnp.float32),
  )(h, parts, deg2, Ws, Wn, b)


def _pred_tc(prod, W1, c1, W2, c2, W3p, c3p):
  m = prod.shape[0]
  bm = 4096

  def body(x_ref, w1, b1, w2, b2, w3, b3, o_ref):
    h1 = jnp.dot(x_ref[...], w1[...], preferred_element_type=jnp.float32)
    h1 = jnp.maximum(h1 + b1[...], 0.0)
    h2 = jnp.dot(h1, w2[...], preferred_element_type=jnp.float32)
    h2 = jnp.maximum(h2 + b2[...], 0.0)
    o_ref[...] = jnp.dot(h2, w3[...], preferred_element_type=jnp.float32) + b3[...]

  return pl.pallas_call(
      body,
      grid=(m // bm,),
      in_specs=[
          pl.BlockSpec((bm, D), lambda i: (i, 0)),
          pl.BlockSpec((D, D), lambda i: (0, 0)),
          pl.BlockSpec((1, D), lambda i: (0, 0)),
          pl.BlockSpec((D, D), lambda i: (0, 0)),
          pl.BlockSpec((1, D), lambda i: (0, 0)),
          pl.BlockSpec((D, 8), lambda i: (0, 0)),
          pl.BlockSpec((1, 8), lambda i: (0, 0)),
      ],
      out_specs=pl.BlockSpec((bm, 8), lambda i: (i, 0)),
      out_shape=jax.ShapeDtypeStruct((m, 8), jnp.float32),
  )(prod, W1, c1, W2, c2, W3p, c3p)


def kernel(x, edge_index, pos_edge_index, neg_edge_index,
           W_self1, W_neigh1, b1, W_self2, W_neigh2, b2,
           W_self3, W_neigh3, b3,
           P1_W, P1_b, P2_W, P2_b, P3_W, P3_b):
  src = edge_index[0].astype(jnp.int32)
  dst = edge_index[1].astype(jnp.int32)

  parts1, = _agg(x, src, dst)
  parts1 = parts1.reshape(NC, N_PAD, D)
  deg2, = _deg(dst)
  deg2 = deg2.reshape(NC, N_PAD, DEGW)
  h1 = _layer_tc(x, parts1, deg2, W_self1, W_neigh1,
                 b1.reshape(1, D), relu=True)
  parts2, = _agg(h1, src, dst)
  parts2 = parts2.reshape(NC, N_PAD, D)
  h2 = _layer_tc(h1, parts2, deg2, W_self2, W_neigh2,
                 b2.reshape(1, D), relu=True)
  parts3, = _agg(h2, src, dst)
  parts3 = parts3.reshape(NC, N_PAD, D)
  h3 = _layer_tc(h2, parts3, deg2, W_self3, W_neigh3,
                 b3.reshape(1, D), relu=False)

  pair = jnp.concatenate([pos_edge_index, neg_edge_index], axis=1)
  a_idx = pair[0].astype(jnp.int32)
  b_idx = pair[1].astype(jnp.int32)
  prod, = _pairs(h3, a_idx, b_idx)

  W3p = jnp.pad(P3_W, ((0, 0), (0, 7)))
  c3p = jnp.pad(P3_b, (0, 7)).reshape(1, 8)
  out = _pred_tc(prod, P1_W, P1_b.reshape(1, D), P2_W, P2_b.reshape(1, D),
                 W3p, c3p)
  col = out[:, 0:1]
  return (col[:N_PAIRS], col[N_PAIRS:])
